# baseline (speedup 1.0000x reference)
import functools

import jax
import jax.numpy as jnp
from jax import lax
from jax.experimental import pallas as pl
from jax.experimental.pallas import tpu as pltpu
from jax.experimental.pallas import tpu_sc as plsc

N = 10000
E = 320000
D = 128
D_EDGE = 16

NC = 2
NS = 16
L = 16

K = 125
EPT = E // NS
CH = EPT // K
SUP = 10
CPS = CH // SUP
NPAD = 10240
DSL = NPAD // NS
NT = N // NS

_mesh = plsc.VectorSubcoreMesh(core_axis_name="c", subcore_axis_name="s")


@functools.partial(
    pl.kernel,
    out_type=(
        jax.ShapeDtypeStruct((NPAD,), jnp.float32),
        jax.ShapeDtypeStruct((NPAD,), jnp.float32),
    ),
    mesh=_mesh,
    scratch_types=[
        pltpu.VMEM((CH, K), jnp.int32),
        pltpu.VMEM((128,), jnp.float32),
        pltpu.VMEM_SHARED((NPAD,), jnp.float32),
        pltpu.SemaphoreType.DMA,
        pltpu.SemaphoreType.DMA,
    ],
)
def _deg_kernel(src_hbm, dst_hbm, zeros_hbm, degout_hbm, degin_hbm,
                idx_v, ones_v, acc_sh, dsem0, dsem1):
    c = lax.axis_index("c")
    s = lax.axis_index("s")
    dsem = (dsem0, dsem1)

    for i in range(128 // L):
        ones_v[pl.ds(i * L, L)] = jnp.full((L,), 1.0, jnp.float32)
    ones_src = ones_v.at[pl.ds(0, K)]

    pltpu.sync_copy(zeros_hbm.at[pl.ds(s * DSL, DSL)],
                    acc_sh.at[pl.ds(s * DSL, DSL)])

    @pl.when(c == 0)
    def _():
        pltpu.sync_copy(src_hbm.at[s], idx_v)

    @pl.when(c == 1)
    def _():
        pltpu.sync_copy(dst_hbm.at[s], idx_v)

    plsc.subcore_barrier()

    sd = [None] * CH
    for j in range(CH):
        if j >= 2:
            sd[j - 2].wait()
        sd[j] = pltpu.async_copy(ones_src, acc_sh.at[idx_v.at[j]],
                                 dsem[j % 2], add=True)
    sd[CH - 2].wait()
    sd[CH - 1].wait()
    plsc.subcore_barrier()

    @pl.when(c == 0)
    def _():
        pltpu.sync_copy(acc_sh.at[pl.ds(s * DSL, DSL)],
                        degout_hbm.at[pl.ds(s * DSL, DSL)])

    @pl.when(c == 1)
    def _():
        pltpu.sync_copy(acc_sh.at[pl.ds(s * DSL, DSL)],
                        degin_hbm.at[pl.ds(s * DSL, DSL)])


@functools.partial(
    pl.kernel,
    out_type=(
        jax.ShapeDtypeStruct((NPAD, D), jnp.float32),
        jax.ShapeDtypeStruct((NPAD, D), jnp.float32),
    ),
    mesh=_mesh,
    scratch_types=[
        pltpu.VMEM((2, CPS, K), jnp.int32),
        pltpu.VMEM((2, CPS, K), jnp.int32),
        pltpu.VMEM((2, K, D), jnp.float32),
        pltpu.VMEM_SHARED((NPAD, D), jnp.float32),
        pltpu.SemaphoreType.DMA,
        pltpu.SemaphoreType.DMA,
        pltpu.SemaphoreType.DMA,
        pltpu.SemaphoreType.DMA,
        pltpu.SemaphoreType.DMA,
        pltpu.SemaphoreType.DMA,
    ],
)
def _scatter_kernel(h_hbm, relu_hbm, src_hbm, dst_hbm, zeros_hbm,
                    a_hbm, r_hbm, sidx_v, didx_v, rows_v, acc_sh,
                    gsem0, gsem1, ssem0, ssem1, isem_s, isem_d):
    c = lax.axis_index("c")
    s = lax.axis_index("s")
    gsem = (gsem0, gsem1)
    ssem = (ssem0, ssem1)

    pltpu.sync_copy(zeros_hbm.at[pl.ds(s * DSL, DSL), :],
                    acc_sh.at[pl.ds(s * DSL, DSL), :])

    plsc.subcore_barrier()

    def pipelined_superstep(gather_fn, didx_bank):
        gd = [None] * CPS
        sd = [None] * CPS
        for j in range(CPS):
            b = j % 2
            if j >= 2:
                sd[j - 2].wait()
            gd[j] = gather_fn(j, rows_v.at[b], gsem[b])
            if j >= 1:
                gd[j - 1].wait()
                sd[j - 1] = pltpu.async_copy(
                    rows_v.at[(j - 1) % 2],
                    acc_sh.at[didx_bank.at[j - 1]],
                    ssem[(j - 1) % 2], add=True)
        gd[CPS - 1].wait()
        sd[CPS - 1] = pltpu.async_copy(
            rows_v.at[(CPS - 1) % 2],
            acc_sh.at[didx_bank.at[CPS - 1]],
            ssem[(CPS - 1) % 2], add=True)
        sd[CPS - 2].wait()
        sd[CPS - 1].wait()

    @pl.when(c == 0)
    def _():
        pltpu.sync_copy(src_hbm.at[s, 0], sidx_v.at[0])
        pltpu.sync_copy(dst_hbm.at[s, 0], didx_v.at[0])

        def sstep(g, carry):
            b = lax.rem(g, 2)

            @pl.when(g > 0)
            def _():
                pltpu.make_async_copy(src_hbm.at[s, g], sidx_v.at[b],
                                      isem_s).wait()
                pltpu.make_async_copy(dst_hbm.at[s, g], didx_v.at[b],
                                      isem_d).wait()

            @pl.when(g + 1 < SUP)
            def _():
                pltpu.async_copy(src_hbm.at[s, g + 1], sidx_v.at[1 - b],
                                 isem_s)
                pltpu.async_copy(dst_hbm.at[s, g + 1], didx_v.at[1 - b],
                                 isem_d)

            def gather(j, buf, sem):
                return pltpu.async_copy(h_hbm.at[sidx_v.at[b, j]], buf, sem)
            pipelined_superstep(gather, didx_v.at[b])
            return carry
        lax.fori_loop(0, SUP, sstep, 0)

    @pl.when(c == 1)
    def _():
        pltpu.sync_copy(dst_hbm.at[s, 0], didx_v.at[0])

        def sstep(g, carry):
            b = lax.rem(g, 2)

            @pl.when(g > 0)
            def _():
                pltpu.make_async_copy(dst_hbm.at[s, g], didx_v.at[b],
                                      isem_d).wait()

            @pl.when(g + 1 < SUP)
            def _():
                pltpu.async_copy(dst_hbm.at[s, g + 1], didx_v.at[1 - b],
                                 isem_d)

            def gather(j, buf, sem):
                cid = (s * SUP + g) * CPS + j
                return pltpu.async_copy(relu_hbm.at[cid], buf, sem)
            pipelined_superstep(gather, didx_v.at[b])
            return carry
        lax.fori_loop(0, SUP, sstep, 0)

    plsc.subcore_barrier()

    @pl.when(c == 0)
    def _():
        pltpu.sync_copy(acc_sh.at[pl.ds(s * DSL, DSL), :],
                        a_hbm.at[pl.ds(s * DSL, DSL), :])

    @pl.when(c == 1)
    def _():
        pltpu.sync_copy(acc_sh.at[pl.ds(s * DSL, DSL), :],
                        r_hbm.at[pl.ds(s * DSL, DSL), :])


def _h_body(x_ref, w_ref, deg_ref, o_ref):
    x = x_ref[...]
    w = w_ref[...]
    norm = lax.rsqrt(jnp.maximum(deg_ref[...], 1.0))
    o_ref[...] = jnp.dot(x, w, preferred_element_type=jnp.float32) * norm


def _h_kernel(x, w, deg):
    bn = 1000
    return pl.pallas_call(
        _h_body,
        grid=(N // bn,),
        in_specs=[
            pl.BlockSpec((bn, D), lambda i: (i, 0)),
            pl.BlockSpec((D, D), lambda i: (0, 0)),
            pl.BlockSpec((bn, 1), lambda i: (i, 0)),
        ],
        out_specs=pl.BlockSpec((bn, D), lambda i: (i, 0)),
        out_shape=jax.ShapeDtypeStruct((N, D), jnp.float32),
    )(x, w, deg)


def _mlp1_body(ef_ref, w1_ref, b1_ref, o_ref):
    y = jnp.dot(ef_ref[...], w1_ref[...], preferred_element_type=jnp.float32)
    o_ref[...] = jnp.maximum(y + b1_ref[...], 0.0)


def _mlp1_kernel(ef, w1, b1):
    be = 8000
    return pl.pallas_call(
        _mlp1_body,
        grid=(E // be,),
        in_specs=[
            pl.BlockSpec((be, D_EDGE), lambda i: (i, 0)),
            pl.BlockSpec((D_EDGE, D), lambda i: (0, 0)),
            pl.BlockSpec((1, D), lambda i: (0, 0)),
        ],
        out_specs=pl.BlockSpec((be, D), lambda i: (i, 0)),
        out_shape=jax.ShapeDtypeStruct((E, D), jnp.float32),
    )(ef, w1, b1)


def _combine_body(a_ref, r_ref, w2_ref, deg_ref, b_ref, b2_ref, o_ref):
    deg = deg_ref[...]
    degc = jnp.maximum(deg, 1.0)
    rw2 = jnp.dot(r_ref[...], w2_ref[...], preferred_element_type=jnp.float32)
    o_ref[...] = (a_ref[...] * lax.rsqrt(degc)
                  + (rw2 + deg * b2_ref[...]) / degc
                  + b_ref[...])


def _combine_kernel(a, r, w2, deg, b, b2):
    bn = 1000
    return pl.pallas_call(
        _combine_body,
        grid=(N // bn,),
        in_specs=[
            pl.BlockSpec((bn, D), lambda i: (i, 0)),
            pl.BlockSpec((bn, D), lambda i: (i, 0)),
            pl.BlockSpec((D, D), lambda i: (0, 0)),
            pl.BlockSpec((bn, 1), lambda i: (i, 0)),
            pl.BlockSpec((1, D), lambda i: (0, 0)),
            pl.BlockSpec((1, D), lambda i: (0, 0)),
        ],
        out_specs=pl.BlockSpec((bn, D), lambda i: (i, 0)),
        out_shape=jax.ShapeDtypeStruct((N, D), jnp.float32),
    )(a, r, w2, deg, b, b2)


def kernel(node_feats, edge_index, edge_feats, W, b, W1, b1, W2, b2):
    src = edge_index[0].reshape(NS, SUP, CPS, K)
    dst = edge_index[1].reshape(NS, SUP, CPS, K)
    src2 = edge_index[0].reshape(NS, CH, K)
    dst2 = edge_index[1].reshape(NS, CH, K)

    zdeg = jnp.zeros((NPAD,), jnp.float32)
    deg_out_p, deg_in_p = _deg_kernel(src2, dst2, zdeg)
    deg_out = deg_out_p[:N].reshape(N, 1)
    deg_in = deg_in_p[:N].reshape(N, 1)

    h = _h_kernel(node_feats, W, deg_out)
    relu_out = _mlp1_kernel(edge_feats, W1, b1.reshape(1, D))

    zacc = jnp.zeros((NPAD, D), jnp.float32)
    agg, rsum = _scatter_kernel(h, relu_out.reshape(E // K, K, D),
                                src, dst, zacc)

    return _combine_kernel(agg[:N], rsum[:N], W2, deg_in, b.reshape(1, D),
                           b2.reshape(1, D))


# speedup vs baseline: 1.2237x; 1.2237x over previous
import functools

import jax
import jax.numpy as jnp
from jax import lax
from jax.experimental import pallas as pl
from jax.experimental.pallas import tpu as pltpu
from jax.experimental.pallas import tpu_sc as plsc

N = 10000
E = 320000
D = 128
D_EDGE = 16

NC = 2
NS = 16
L = 16

K = 125
EPT = E // NS
CH = EPT // K
SUP = 10
CPS = CH // SUP
K1 = 80
CH1 = EPT // K1
CPS1 = CH1 // SUP
NPAD = 10240
DSL = NPAD // NS
NT = N // NS

_mesh = plsc.VectorSubcoreMesh(core_axis_name="c", subcore_axis_name="s")


@functools.partial(
    pl.kernel,
    out_type=(
        jax.ShapeDtypeStruct((NPAD,), jnp.float32),
        jax.ShapeDtypeStruct((NPAD,), jnp.float32),
    ),
    mesh=_mesh,
    scratch_types=[
        pltpu.VMEM((CH, K), jnp.int32),
        pltpu.VMEM((128,), jnp.float32),
        pltpu.VMEM_SHARED((NPAD,), jnp.float32),
        pltpu.SemaphoreType.DMA,
        pltpu.SemaphoreType.DMA,
    ],
)
def _deg_kernel(src_hbm, dst_hbm, zeros_hbm, degout_hbm, degin_hbm,
                idx_v, ones_v, acc_sh, dsem0, dsem1):
    c = lax.axis_index("c")
    s = lax.axis_index("s")
    dsem = (dsem0, dsem1)

    for i in range(128 // L):
        ones_v[pl.ds(i * L, L)] = jnp.full((L,), 1.0, jnp.float32)
    ones_src = ones_v.at[pl.ds(0, K)]

    pltpu.sync_copy(zeros_hbm.at[pl.ds(s * DSL, DSL)],
                    acc_sh.at[pl.ds(s * DSL, DSL)])

    @pl.when(c == 0)
    def _():
        pltpu.sync_copy(src_hbm.at[s], idx_v)

    @pl.when(c == 1)
    def _():
        pltpu.sync_copy(dst_hbm.at[s], idx_v)

    plsc.subcore_barrier()

    sd = [None] * CH
    for j in range(CH):
        if j >= 2:
            sd[j - 2].wait()
        sd[j] = pltpu.async_copy(ones_src, acc_sh.at[idx_v.at[j]],
                                 dsem[j % 2], add=True)
    sd[CH - 2].wait()
    sd[CH - 1].wait()
    plsc.subcore_barrier()

    @pl.when(c == 0)
    def _():
        pltpu.sync_copy(acc_sh.at[pl.ds(s * DSL, DSL)],
                        degout_hbm.at[pl.ds(s * DSL, DSL)])

    @pl.when(c == 1)
    def _():
        pltpu.sync_copy(acc_sh.at[pl.ds(s * DSL, DSL)],
                        degin_hbm.at[pl.ds(s * DSL, DSL)])


@functools.partial(
    pl.kernel,
    out_type=(
        jax.ShapeDtypeStruct((NPAD, D), jnp.float32),
        jax.ShapeDtypeStruct((NPAD, D), jnp.float32),
    ),
    mesh=_mesh,
    scratch_types=[
        pltpu.VMEM((2, CPS, K), jnp.int32),
        pltpu.VMEM((2, CPS, K), jnp.int32),
        pltpu.VMEM((2, CPS1, K1), jnp.int32),
        pltpu.VMEM((2, K, D), jnp.float32),
        pltpu.VMEM_SHARED((NPAD, D), jnp.float32),
        pltpu.SemaphoreType.DMA,
        pltpu.SemaphoreType.DMA,
        pltpu.SemaphoreType.DMA,
        pltpu.SemaphoreType.DMA,
        pltpu.SemaphoreType.DMA,
        pltpu.SemaphoreType.DMA,
    ],
)
def _scatter_kernel(h_hbm, relu_hbm, src_hbm, dst_hbm, dst1_hbm, zeros_hbm,
                    a_hbm, r_hbm, sidx_v, didx_v, didx1_v, rows_v, acc_sh,
                    gsem0, gsem1, ssem0, ssem1, isem_s, isem_d):
    c = lax.axis_index("c")
    s = lax.axis_index("s")
    gsem = (gsem0, gsem1)
    ssem = (ssem0, ssem1)

    pltpu.sync_copy(zeros_hbm.at[pl.ds(s * DSL, DSL), :],
                    acc_sh.at[pl.ds(s * DSL, DSL), :])

    plsc.subcore_barrier()

    def pipelined_superstep(cps, gather_fn, scat_src_fn, didx_bank):
        gd = [None] * cps
        sd = [None] * cps

        def scat(j):
            return pltpu.async_copy(scat_src_fn(j % 2),
                                    acc_sh.at[didx_bank.at[j]],
                                    ssem[j % 2], add=True)

        for j in range(cps):
            b = j % 2
            if j >= 2:
                sd[j - 2].wait()
            gd[j] = gather_fn(j, b)
            if j >= 1:
                gd[j - 1].wait()
                sd[j - 1] = scat(j - 1)
        gd[cps - 1].wait()
        sd[cps - 1] = scat(cps - 1)
        sd[cps - 2].wait()
        sd[cps - 1].wait()

    @pl.when(c == 0)
    def _():
        pltpu.sync_copy(src_hbm.at[s, 0], sidx_v.at[0])
        pltpu.sync_copy(dst_hbm.at[s, 0], didx_v.at[0])

        def sstep(g, carry):
            b = lax.rem(g, 2)

            @pl.when(g > 0)
            def _():
                pltpu.make_async_copy(src_hbm.at[s, g], sidx_v.at[b],
                                      isem_s).wait()
                pltpu.make_async_copy(dst_hbm.at[s, g], didx_v.at[b],
                                      isem_d).wait()

            @pl.when(g + 1 < SUP)
            def _():
                pltpu.async_copy(src_hbm.at[s, g + 1], sidx_v.at[1 - b],
                                 isem_s)
                pltpu.async_copy(dst_hbm.at[s, g + 1], didx_v.at[1 - b],
                                 isem_d)

            def gather(j, rb):
                return pltpu.async_copy(h_hbm.at[sidx_v.at[b, j]],
                                        rows_v.at[rb], gsem[rb])
            pipelined_superstep(CPS, gather, lambda rb: rows_v.at[rb],
                                didx_v.at[b])
            return carry
        lax.fori_loop(0, SUP, sstep, 0)

    @pl.when(c == 1)
    def _():
        pltpu.sync_copy(dst1_hbm.at[s, 0], didx1_v.at[0])

        def sstep(g, carry):
            b = lax.rem(g, 2)

            @pl.when(g > 0)
            def _():
                pltpu.make_async_copy(dst1_hbm.at[s, g], didx1_v.at[b],
                                      isem_d).wait()

            @pl.when(g + 1 < SUP)
            def _():
                pltpu.async_copy(dst1_hbm.at[s, g + 1], didx1_v.at[1 - b],
                                 isem_d)

            def gather(j, rb):
                base = ((s * SUP + g) * CPS1 + j) * K1
                return pltpu.async_copy(relu_hbm.at[pl.ds(base, K1), :],
                                        rows_v.at[rb, pl.ds(0, K1)],
                                        gsem[rb])
            pipelined_superstep(CPS1, gather,
                                lambda rb: rows_v.at[rb, pl.ds(0, K1)],
                                didx1_v.at[b])
            return carry
        lax.fori_loop(0, SUP, sstep, 0)

    plsc.subcore_barrier()

    @pl.when(c == 0)
    def _():
        pltpu.sync_copy(acc_sh.at[pl.ds(s * DSL, DSL), :],
                        a_hbm.at[pl.ds(s * DSL, DSL), :])

    @pl.when(c == 1)
    def _():
        pltpu.sync_copy(acc_sh.at[pl.ds(s * DSL, DSL), :],
                        r_hbm.at[pl.ds(s * DSL, DSL), :])


def _h_body(x_ref, w_ref, deg_ref, o_ref):
    x = x_ref[...]
    w = w_ref[...]
    norm = lax.rsqrt(jnp.maximum(deg_ref[...], 1.0))
    o_ref[...] = jnp.dot(x, w, preferred_element_type=jnp.float32) * norm


def _h_kernel(x, w, deg):
    bn = 1000
    return pl.pallas_call(
        _h_body,
        grid=(N // bn,),
        in_specs=[
            pl.BlockSpec((bn, D), lambda i: (i, 0)),
            pl.BlockSpec((D, D), lambda i: (0, 0)),
            pl.BlockSpec((bn, 1), lambda i: (i, 0)),
        ],
        out_specs=pl.BlockSpec((bn, D), lambda i: (i, 0)),
        out_shape=jax.ShapeDtypeStruct((N, D), jnp.float32),
    )(x, w, deg)


def _mlp1_body(ef_ref, w1_ref, b1_ref, o_ref):
    y = jnp.dot(ef_ref[...], w1_ref[...], preferred_element_type=jnp.float32)
    o_ref[...] = jnp.maximum(y + b1_ref[...], 0.0)


def _mlp1_kernel(ef, w1, b1):
    be = 8000
    return pl.pallas_call(
        _mlp1_body,
        grid=(E // be,),
        in_specs=[
            pl.BlockSpec((be, D_EDGE), lambda i: (i, 0)),
            pl.BlockSpec((D_EDGE, D), lambda i: (0, 0)),
            pl.BlockSpec((1, D), lambda i: (0, 0)),
        ],
        out_specs=pl.BlockSpec((be, D), lambda i: (i, 0)),
        out_shape=jax.ShapeDtypeStruct((E, D), jnp.float32),
    )(ef, w1, b1)


def _combine_body(a_ref, r_ref, w2_ref, deg_ref, b_ref, b2_ref, o_ref):
    deg = deg_ref[...]
    degc = jnp.maximum(deg, 1.0)
    rw2 = jnp.dot(r_ref[...], w2_ref[...], preferred_element_type=jnp.float32)
    o_ref[...] = (a_ref[...] * lax.rsqrt(degc)
                  + (rw2 + deg * b2_ref[...]) / degc
                  + b_ref[...])


def _combine_kernel(a, r, w2, deg, b, b2):
    bn = 1000
    return pl.pallas_call(
        _combine_body,
        grid=(N // bn,),
        in_specs=[
            pl.BlockSpec((bn, D), lambda i: (i, 0)),
            pl.BlockSpec((bn, D), lambda i: (i, 0)),
            pl.BlockSpec((D, D), lambda i: (0, 0)),
            pl.BlockSpec((bn, 1), lambda i: (i, 0)),
            pl.BlockSpec((1, D), lambda i: (0, 0)),
            pl.BlockSpec((1, D), lambda i: (0, 0)),
        ],
        out_specs=pl.BlockSpec((bn, D), lambda i: (i, 0)),
        out_shape=jax.ShapeDtypeStruct((N, D), jnp.float32),
    )(a, r, w2, deg, b, b2)


def kernel(node_feats, edge_index, edge_feats, W, b, W1, b1, W2, b2):
    src = edge_index[0].reshape(NS, SUP, CPS, K)
    dst = edge_index[1].reshape(NS, SUP, CPS, K)
    dst1 = edge_index[1].reshape(NS, SUP, CPS1, K1)
    src2 = edge_index[0].reshape(NS, CH, K)
    dst2 = edge_index[1].reshape(NS, CH, K)

    zdeg = jnp.zeros((NPAD,), jnp.float32)
    deg_out_p, deg_in_p = _deg_kernel(src2, dst2, zdeg)
    deg_out = deg_out_p[:N].reshape(N, 1)
    deg_in = deg_in_p[:N].reshape(N, 1)

    h = _h_kernel(node_feats, W, deg_out)
    relu_out = _mlp1_kernel(edge_feats, W1, b1.reshape(1, D))

    zacc = jnp.zeros((NPAD, D), jnp.float32)
    agg, rsum = _scatter_kernel(h, relu_out, src, dst, dst1, zacc)

    return _combine_kernel(agg[:N], rsum[:N], W2, deg_in, b.reshape(1, D),
                           b2.reshape(1, D))
